# baseline (speedup 1.0000x reference)
import functools

import jax
import jax.numpy as jnp
from jax import lax
from jax.experimental import pallas as pl
from jax.experimental.pallas import tpu as pltpu
from jax.experimental.pallas import tpu_sc as plsc

LN_EPS = 1e-12
CHUNK = 128


def _sc_gather(ids2d, table):
    n_chunks, _ = ids2d.shape
    _, emb = table.shape
    info = plsc.get_sparse_core_info()
    nc, ns = info.num_cores, info.num_subcores
    nw = nc * ns
    assert n_chunks % (2 * nw) == 0, "need an even number of chunks per worker"
    cpw = n_chunks // nw
    n_tok = n_chunks * CHUNK
    ids3d = ids2d.reshape(nw, cpw, CHUNK)

    mesh = plsc.VectorSubcoreMesh(core_axis_name="c", subcore_axis_name="s")

    @functools.partial(
        pl.kernel,
        mesh=mesh,
        out_type=jax.ShapeDtypeStruct((n_tok, emb), jnp.float32),
        scratch_types=[
            pltpu.VMEM((cpw, CHUNK), jnp.int32),
            pltpu.VMEM((CHUNK, emb), jnp.float32),
            pltpu.VMEM((CHUNK, emb), jnp.float32),
            pltpu.SemaphoreType.DMA,
            pltpu.SemaphoreType.DMA,
        ],
    )
    def gather_kernel(ids_hbm, table_hbm, out_hbm, idx_v, buf0, buf1, sem0, sem1):
        wid = lax.axis_index("s") * nc + lax.axis_index("c")
        row0 = wid * cpw * CHUNK
        pltpu.sync_copy(ids_hbm.at[wid], idx_v)

        def gather(j, buf, sem):
            return pltpu.make_async_copy(table_hbm.at[idx_v.at[j]], buf, sem)

        gather(0, buf0, sem0).start()

        def body(i, carry):
            j2 = 2 * i
            gather(j2 + 1, buf1, sem1).start()
            gather(j2, buf0, sem0).wait()
            pltpu.sync_copy(buf0, out_hbm.at[pl.ds(row0 + j2 * CHUNK, CHUNK)])

            @pl.when(j2 + 2 < cpw)
            def _():
                gather(j2 + 2, buf0, sem0).start()

            gather(j2 + 1, buf1, sem1).wait()
            pltpu.sync_copy(buf1, out_hbm.at[pl.ds(row0 + (j2 + 1) * CHUNK, CHUNK)])
            return carry

        lax.fori_loop(0, cpw // 2, body, 0)

    return gather_kernel(ids3d, table)


def _tc_fused(x3, pos, w, b):
    bsz, seq, emb = x3.shape
    hid = w.shape[1]

    def body(x_ref, pos_ref, w_ref, b_ref, o_ref):
        x = x_ref[0] + pos_ref[...]
        mean = jnp.mean(x, axis=-1, keepdims=True)
        xc = x - mean
        var = jnp.mean(xc * xc, axis=-1, keepdims=True)
        y = (xc * lax.rsqrt(var + LN_EPS)).astype(jnp.bfloat16)
        o_ref[0] = (
            jnp.dot(y, w_ref[...], preferred_element_type=jnp.float32) + b_ref[...]
        )

    return pl.pallas_call(
        body,
        grid=(bsz,),
        in_specs=[
            pl.BlockSpec((1, seq, emb), lambda i: (i, 0, 0)),
            pl.BlockSpec((seq, emb), lambda i: (0, 0)),
            pl.BlockSpec((emb, hid), lambda i: (0, 0)),
            pl.BlockSpec((1, hid), lambda i: (0, 0)),
        ],
        out_specs=pl.BlockSpec((1, seq, hid), lambda i: (i, 0, 0)),
        out_shape=jax.ShapeDtypeStruct((bsz, seq, hid), jnp.float32),
    )(x3, pos, w, b)


def kernel(input_ids, word_emb, pos_emb, ln_scale, ln_bias, kernel, bias):
    bsz, seq = input_ids.shape
    emb = word_emb.shape[1]
    hid = kernel.shape[1]

    ids2d = input_ids.reshape(-1, CHUNK).astype(jnp.int32)
    gathered = _sc_gather(ids2d, word_emb)

    w2 = (ln_scale[:, None] * kernel).astype(jnp.bfloat16)
    b2 = (ln_bias @ kernel + bias)[None, :]
    pos = pos_emb[:seq]

    out = _tc_fused(gathered.reshape(bsz, seq, emb), pos, w2, b2)
    return out


# speedup vs baseline: 1.5322x; 1.5322x over previous
import functools

import jax
import jax.numpy as jnp
from jax import lax
from jax.experimental import pallas as pl
from jax.experimental.pallas import tpu as pltpu
from jax.experimental.pallas import tpu_sc as plsc

LN_EPS = 1e-12
CHUNK = 128


def _sc_gather(ids2d, table):
    n_chunks, _ = ids2d.shape
    _, emb = table.shape
    info = plsc.get_sparse_core_info()
    nc, ns = info.num_cores, info.num_subcores
    nw = nc * ns
    assert n_chunks % (2 * nw) == 0, "need an even number of chunks per worker"
    cpw = n_chunks // nw
    n_tok = n_chunks * CHUNK
    ids3d = ids2d.reshape(nw, cpw, CHUNK)

    mesh = plsc.VectorSubcoreMesh(core_axis_name="c", subcore_axis_name="s")

    @functools.partial(
        pl.kernel,
        mesh=mesh,
        out_type=jax.ShapeDtypeStruct((n_tok, emb), jnp.float32),
        scratch_types=[
            pltpu.VMEM((cpw, CHUNK), jnp.int32),
            pltpu.VMEM((CHUNK, emb), jnp.float32),
            pltpu.VMEM((CHUNK, emb), jnp.float32),
            pltpu.SemaphoreType.DMA,
            pltpu.SemaphoreType.DMA,
        ],
    )
    def gather_kernel(ids_hbm, table_hbm, out_hbm, idx_v, buf0, buf1, sem0, sem1):
        wid = lax.axis_index("s") * nc + lax.axis_index("c")
        row0 = wid * cpw * CHUNK
        pltpu.sync_copy(ids_hbm.at[wid], idx_v)

        def gather(j, buf, sem):
            return pltpu.make_async_copy(table_hbm.at[idx_v.at[j]], buf, sem)

        gather(0, buf0, sem0).start()

        def body(i, carry):
            j2 = 2 * i
            gather(j2 + 1, buf1, sem1).start()
            gather(j2, buf0, sem0).wait()
            pltpu.sync_copy(buf0, out_hbm.at[pl.ds(row0 + j2 * CHUNK, CHUNK)])

            @pl.when(j2 + 2 < cpw)
            def _():
                gather(j2 + 2, buf0, sem0).start()

            gather(j2 + 1, buf1, sem1).wait()
            pltpu.sync_copy(buf1, out_hbm.at[pl.ds(row0 + (j2 + 1) * CHUNK, CHUNK)])
            return carry

        lax.fori_loop(0, cpw // 2, body, 0)

    return gather_kernel(ids3d, table)


def _tc_fused(x3, pos, w, b, rows_per_step=2):
    bsz, seq, emb = x3.shape
    hid = w.shape[1]
    r = rows_per_step
    assert bsz % r == 0

    def body(x_ref, pos_ref, w_ref, b_ref, o_ref):
        x = x_ref[...] + pos_ref[...]
        mean = jnp.mean(x, axis=-1, keepdims=True)
        msq = jnp.mean(x * x, axis=-1, keepdims=True)
        var = msq - mean * mean
        y = (x - mean) * lax.rsqrt(var + LN_EPS)
        res = jnp.dot(
            y.reshape(r * seq, emb), w_ref[...], preferred_element_type=jnp.float32
        )
        o_ref[...] = res.reshape(r, seq, hid) + b_ref[...]

    return pl.pallas_call(
        body,
        grid=(bsz // r,),
        in_specs=[
            pl.BlockSpec((r, seq, emb), lambda i: (i, 0, 0)),
            pl.BlockSpec((1, seq, emb), lambda i: (0, 0, 0)),
            pl.BlockSpec((emb, hid), lambda i: (0, 0)),
            pl.BlockSpec((1, 1, hid), lambda i: (0, 0, 0)),
        ],
        out_specs=pl.BlockSpec((r, seq, hid), lambda i: (i, 0, 0)),
        out_shape=jax.ShapeDtypeStruct((bsz, seq, hid), jnp.float32),
    )(x3, pos[None], w, b[None])


def kernel(input_ids, word_emb, pos_emb, ln_scale, ln_bias, kernel, bias):
    bsz, seq = input_ids.shape
    emb = word_emb.shape[1]
    hid = kernel.shape[1]

    ids2d = input_ids.reshape(-1, CHUNK).astype(jnp.int32)
    gathered = _sc_gather(ids2d, word_emb)

    w2 = ln_scale[:, None] * kernel
    b2 = (ln_bias @ kernel + bias)[None, :]
    pos = pos_emb[:seq]

    out = _tc_fused(gathered.reshape(bsz, seq, emb), pos, w2, b2)
    return out


# speedup vs baseline: 2.0121x; 1.3132x over previous
import functools

import jax
import jax.numpy as jnp
from jax import lax
from jax.experimental import pallas as pl
from jax.experimental.pallas import tpu as pltpu
from jax.experimental.pallas import tpu_sc as plsc

LN_EPS = 1e-12
CHUNK = 128


def _sc_gather(ids2d, table):
    n_chunks, _ = ids2d.shape
    _, emb = table.shape
    info = plsc.get_sparse_core_info()
    nc, ns = info.num_cores, info.num_subcores
    nw = nc * ns
    assert n_chunks % (2 * nw) == 0, "need an even number of chunks per worker"
    cpw = n_chunks // nw
    n_tok = n_chunks * CHUNK
    ids3d = ids2d.reshape(nw, cpw, CHUNK)

    mesh = plsc.VectorSubcoreMesh(core_axis_name="c", subcore_axis_name="s")

    @functools.partial(
        pl.kernel,
        mesh=mesh,
        out_type=jax.ShapeDtypeStruct((n_tok, emb), jnp.float32),
        scratch_types=[
            pltpu.VMEM((cpw, CHUNK), jnp.int32),
            pltpu.VMEM((CHUNK, emb), jnp.float32),
            pltpu.VMEM((CHUNK, emb), jnp.float32),
            pltpu.SemaphoreType.DMA,
            pltpu.SemaphoreType.DMA,
        ],
    )
    def gather_kernel(ids_hbm, table_hbm, out_hbm, idx_v, buf0, buf1, sem0, sem1):
        wid = lax.axis_index("s") * nc + lax.axis_index("c")
        row0 = wid * cpw * CHUNK
        pltpu.sync_copy(ids_hbm.at[wid], idx_v)

        def gather(j, buf, sem):
            return pltpu.make_async_copy(table_hbm.at[idx_v.at[j]], buf, sem)

        gather(0, buf0, sem0).start()

        def body(i, carry):
            j2 = 2 * i
            gather(j2 + 1, buf1, sem1).start()
            gather(j2, buf0, sem0).wait()
            pltpu.sync_copy(buf0, out_hbm.at[pl.ds(row0 + j2 * CHUNK, CHUNK)])

            @pl.when(j2 + 2 < cpw)
            def _():
                gather(j2 + 2, buf0, sem0).start()

            gather(j2 + 1, buf1, sem1).wait()
            pltpu.sync_copy(buf1, out_hbm.at[pl.ds(row0 + (j2 + 1) * CHUNK, CHUNK)])
            return carry

        lax.fori_loop(0, cpw // 2, body, 0)

    return gather_kernel(ids3d, table)


def _tc_fused(x3, pos, w, b, rows_per_step=4):
    bsz, seq, emb = x3.shape
    hid = w.shape[1]
    r = rows_per_step
    assert bsz % r == 0

    def body(x_ref, pos_ref, w_ref, b_ref, o_ref):
        x = x_ref[...] + pos_ref[...]
        mean = jnp.mean(x, axis=-1, keepdims=True)
        msq = jnp.mean(x * x, axis=-1, keepdims=True)
        var = msq - mean * mean
        y = (x - mean) * lax.rsqrt(var + LN_EPS)
        res = jnp.dot(
            y.reshape(r * seq, emb), w_ref[...], preferred_element_type=jnp.float32
        )
        o_ref[...] = res.reshape(r, seq, hid) + b_ref[...]

    return pl.pallas_call(
        body,
        grid=(bsz // r,),
        in_specs=[
            pl.BlockSpec((r, seq, emb), lambda i: (i, 0, 0)),
            pl.BlockSpec((1, seq, emb), lambda i: (0, 0, 0)),
            pl.BlockSpec((emb, hid), lambda i: (0, 0)),
            pl.BlockSpec((1, 1, hid), lambda i: (0, 0, 0)),
        ],
        out_specs=pl.BlockSpec((r, seq, hid), lambda i: (i, 0, 0)),
        out_shape=jax.ShapeDtypeStruct((bsz, seq, hid), jnp.float32),
    )(x3, pos[None], w, b[None])


def kernel(input_ids, word_emb, pos_emb, ln_scale, ln_bias, kernel, bias):
    bsz, seq = input_ids.shape
    emb = word_emb.shape[1]
    hid = kernel.shape[1]

    ids2d = input_ids.reshape(-1, CHUNK).astype(jnp.int32)
    gathered = _sc_gather(ids2d, word_emb)

    w2 = ln_scale[:, None] * kernel
    b2 = (ln_bias @ kernel + bias)[None, :]
    pos = pos_emb[:seq]

    out = _tc_fused(gathered.reshape(bsz, seq, emb), pos, w2, b2)
    return out


# speedup vs baseline: 2.4054x; 1.1955x over previous
import functools

import jax
import jax.numpy as jnp
from jax import lax
from jax.experimental import pallas as pl
from jax.experimental.pallas import tpu as pltpu
from jax.experimental.pallas import tpu_sc as plsc

LN_EPS = 1e-12
CHUNK = 128


def _sc_gather(ids2d, table):
    n_chunks, _ = ids2d.shape
    _, emb = table.shape
    info = plsc.get_sparse_core_info()
    nc, ns = info.num_cores, info.num_subcores
    nw = nc * ns
    assert n_chunks % (2 * nw) == 0, "need an even number of chunks per worker"
    cpw = n_chunks // nw
    n_tok = n_chunks * CHUNK
    ids3d = ids2d.reshape(nw, cpw, CHUNK)

    mesh = plsc.VectorSubcoreMesh(core_axis_name="c", subcore_axis_name="s")

    @functools.partial(
        pl.kernel,
        mesh=mesh,
        out_type=jax.ShapeDtypeStruct((n_tok, emb), jnp.float32),
        scratch_types=[
            pltpu.VMEM((cpw, CHUNK), jnp.int32),
            pltpu.VMEM((CHUNK, emb), jnp.float32),
            pltpu.VMEM((CHUNK, emb), jnp.float32),
            pltpu.SemaphoreType.DMA,
            pltpu.SemaphoreType.DMA,
        ],
    )
    def gather_kernel(ids_hbm, table_hbm, out_hbm, idx_v, buf0, buf1, sem0, sem1):
        wid = lax.axis_index("s") * nc + lax.axis_index("c")
        row0 = wid * cpw * CHUNK
        pltpu.sync_copy(ids_hbm.at[wid], idx_v)

        def gather(j, buf, sem):
            return pltpu.make_async_copy(table_hbm.at[idx_v.at[j]], buf, sem)

        gather(0, buf0, sem0).start()

        def body(i, carry):
            j2 = 2 * i
            gather(j2 + 1, buf1, sem1).start()
            gather(j2, buf0, sem0).wait()
            pltpu.sync_copy(buf0, out_hbm.at[pl.ds(row0 + j2 * CHUNK, CHUNK)])

            @pl.when(j2 + 2 < cpw)
            def _():
                gather(j2 + 2, buf0, sem0).start()

            gather(j2 + 1, buf1, sem1).wait()
            pltpu.sync_copy(buf1, out_hbm.at[pl.ds(row0 + (j2 + 1) * CHUNK, CHUNK)])
            return carry

        lax.fori_loop(0, cpw // 2, body, 0)

    return gather_kernel(ids3d, table)


def _tc_fused(x3, pos, w, b, rows_per_step=8):
    bsz, seq, emb = x3.shape
    hid = w.shape[1]
    r = rows_per_step
    assert bsz % r == 0

    def body(x_ref, pos_ref, w_ref, b_ref, o_ref):
        x = x_ref[...] + pos_ref[...]
        mean = jnp.mean(x, axis=-1, keepdims=True)
        msq = jnp.mean(x * x, axis=-1, keepdims=True)
        var = msq - mean * mean
        y = (x - mean) * lax.rsqrt(var + LN_EPS)
        res = jnp.dot(
            y.reshape(r * seq, emb), w_ref[...], preferred_element_type=jnp.float32
        )
        o_ref[...] = res.reshape(r, seq, hid) + b_ref[...]

    return pl.pallas_call(
        body,
        grid=(bsz // r,),
        in_specs=[
            pl.BlockSpec((r, seq, emb), lambda i: (i, 0, 0)),
            pl.BlockSpec((1, seq, emb), lambda i: (0, 0, 0)),
            pl.BlockSpec((emb, hid), lambda i: (0, 0)),
            pl.BlockSpec((1, 1, hid), lambda i: (0, 0, 0)),
        ],
        out_specs=pl.BlockSpec((r, seq, hid), lambda i: (i, 0, 0)),
        out_shape=jax.ShapeDtypeStruct((bsz, seq, hid), jnp.float32),
    )(x3, pos[None], w, b[None])


def kernel(input_ids, word_emb, pos_emb, ln_scale, ln_bias, kernel, bias):
    bsz, seq = input_ids.shape
    emb = word_emb.shape[1]
    hid = kernel.shape[1]

    ids2d = input_ids.reshape(-1, CHUNK).astype(jnp.int32)
    gathered = _sc_gather(ids2d, word_emb)

    w2 = ln_scale[:, None] * kernel
    b2 = (ln_bias @ kernel + bias)[None, :]
    pos = pos_emb[:seq]

    out = _tc_fused(gathered.reshape(bsz, seq, emb), pos, w2, b2)
    return out


# speedup vs baseline: 2.4698x; 1.0268x over previous
import functools

import jax
import jax.numpy as jnp
from jax import lax
from jax.experimental import pallas as pl
from jax.experimental.pallas import tpu as pltpu
from jax.experimental.pallas import tpu_sc as plsc

LN_EPS = 1e-12
CHUNK = 128


def _sc_gather(ids2d, table):
    n_chunks, _ = ids2d.shape
    _, emb = table.shape
    info = plsc.get_sparse_core_info()
    nc, ns = info.num_cores, info.num_subcores
    nw = nc * ns
    assert n_chunks % (2 * nw) == 0, "need an even number of chunks per worker"
    cpw = n_chunks // nw
    n_tok = n_chunks * CHUNK
    ids3d = ids2d.reshape(nw, cpw, CHUNK)

    mesh = plsc.VectorSubcoreMesh(core_axis_name="c", subcore_axis_name="s")

    @functools.partial(
        pl.kernel,
        mesh=mesh,
        out_type=jax.ShapeDtypeStruct((n_tok, emb), jnp.float32),
        scratch_types=[
            pltpu.VMEM((cpw, CHUNK), jnp.int32),
            pltpu.VMEM((CHUNK, emb), jnp.float32),
            pltpu.VMEM((CHUNK, emb), jnp.float32),
            pltpu.SemaphoreType.DMA,
            pltpu.SemaphoreType.DMA,
        ],
    )
    def gather_kernel(ids_hbm, table_hbm, out_hbm, idx_v, buf0, buf1, sem0, sem1):
        wid = lax.axis_index("s") * nc + lax.axis_index("c")
        row0 = wid * cpw * CHUNK
        pltpu.sync_copy(ids_hbm.at[wid], idx_v)

        def gather(j, buf, sem):
            return pltpu.make_async_copy(table_hbm.at[idx_v.at[j]], buf, sem)

        gather(0, buf0, sem0).start()

        def body(i, carry):
            j2 = 2 * i
            gather(j2 + 1, buf1, sem1).start()
            gather(j2, buf0, sem0).wait()
            pltpu.sync_copy(buf0, out_hbm.at[pl.ds(row0 + j2 * CHUNK, CHUNK)])

            @pl.when(j2 + 2 < cpw)
            def _():
                gather(j2 + 2, buf0, sem0).start()

            gather(j2 + 1, buf1, sem1).wait()
            pltpu.sync_copy(buf1, out_hbm.at[pl.ds(row0 + (j2 + 1) * CHUNK, CHUNK)])
            return carry

        lax.fori_loop(0, cpw // 2, body, 0)

    return gather_kernel(ids3d, table)


def _tc_fused(x3, pos, w, b, rows_per_step=16):
    bsz, seq, emb = x3.shape
    hid = w.shape[1]
    r = rows_per_step
    assert bsz % r == 0

    def body(x_ref, pos_ref, w_ref, b_ref, o_ref):
        x = x_ref[...] + pos_ref[...]
        mean = jnp.mean(x, axis=-1, keepdims=True)
        msq = jnp.mean(x * x, axis=-1, keepdims=True)
        var = msq - mean * mean
        y = (x - mean) * lax.rsqrt(var + LN_EPS)
        res = jnp.dot(
            y.reshape(r * seq, emb), w_ref[...], preferred_element_type=jnp.float32
        )
        o_ref[...] = res.reshape(r, seq, hid) + b_ref[...]

    return pl.pallas_call(
        body,
        grid=(bsz // r,),
        in_specs=[
            pl.BlockSpec((r, seq, emb), lambda i: (i, 0, 0)),
            pl.BlockSpec((1, seq, emb), lambda i: (0, 0, 0)),
            pl.BlockSpec((emb, hid), lambda i: (0, 0)),
            pl.BlockSpec((1, 1, hid), lambda i: (0, 0, 0)),
        ],
        out_specs=pl.BlockSpec((r, seq, hid), lambda i: (i, 0, 0)),
        out_shape=jax.ShapeDtypeStruct((bsz, seq, hid), jnp.float32),
    )(x3, pos[None], w, b[None])


def kernel(input_ids, word_emb, pos_emb, ln_scale, ln_bias, kernel, bias):
    bsz, seq = input_ids.shape
    emb = word_emb.shape[1]
    hid = kernel.shape[1]

    ids2d = input_ids.reshape(-1, CHUNK).astype(jnp.int32)
    gathered = _sc_gather(ids2d, word_emb)

    w2 = ln_scale[:, None] * kernel
    b2 = (ln_bias @ kernel + bias)[None, :]
    pos = pos_emb[:seq]

    out = _tc_fused(gathered.reshape(bsz, seq, emb), pos, w2, b2)
    return out
